# baseline (speedup 1.0000x reference)
import jax
import jax.numpy as jnp
from jax.experimental import pallas as pl
from jax.experimental.pallas import tpu as pltpu
from jax.experimental.pallas import tpu_sc as plsc

_RB = 8
_CB = 256
_V = 16


def _sc_add(concat_hbm, table_hbm, out_hbm):
    rows, hidden = concat_hbm.shape
    seq_blocks = table_hbm.shape[0] // _RB

    def body(c_vmem, t_vmem, o_vmem):
        @pl.loop(0, _RB)
        def _(r):
            @pl.loop(0, _CB, step=_V)
            def _(c):
                slc = (r, pl.ds(c, _V))
                o_vmem.at[*slc][...] = c_vmem.at[*slc][...] + t_vmem.at[*slc][...]

    pltpu.emit_pipeline(
        body,
        grid=(rows // _RB, hidden // _CB),
        in_specs=[
            pl.BlockSpec((_RB, _CB), index_map=lambda i, j: (i, j)),
            pl.BlockSpec((_RB, _CB), index_map=lambda i, j: (i % seq_blocks, j)),
        ],
        out_specs=[pl.BlockSpec((_RB, _CB), index_map=lambda i, j: (i, j))],
        core_axis_name=("core", "subcore"),
        dimension_semantics=(pltpu.PARALLEL, pltpu.PARALLEL),
    )(concat_hbm, table_hbm, out_hbm)


def kernel(concat_embeddings, position_table):
    batch, seq, hidden = concat_embeddings.shape
    flat = concat_embeddings.reshape(batch * seq, hidden)
    table = position_table[:seq]

    mesh = plsc.VectorSubcoreMesh(core_axis_name="core", subcore_axis_name="subcore")
    sc_fn = pl.kernel(
        _sc_add,
        out_type=jax.ShapeDtypeStruct((batch * seq, hidden), concat_embeddings.dtype),
        mesh=mesh,
        scratch_types=[],
    )
    out = sc_fn(flat, table)
    return out.reshape(batch, seq, hidden)


# speedup vs baseline: 3.1597x; 3.1597x over previous
import jax
import jax.numpy as jnp
from jax.experimental import pallas as pl


def _add_body(concat_ref, table_ref, out_ref):
    out_ref[...] = concat_ref[...] + table_ref[...][None, :, :]


def kernel(concat_embeddings, position_table):
    batch, seq, hidden = concat_embeddings.shape
    bs = 128
    grid = (seq // bs,)
    table = position_table[:seq]
    return pl.pallas_call(
        _add_body,
        grid=grid,
        in_specs=[
            pl.BlockSpec((batch, bs, hidden), lambda i: (0, i, 0)),
            pl.BlockSpec((bs, hidden), lambda i: (i, 0)),
        ],
        out_specs=pl.BlockSpec((batch, bs, hidden), lambda i: (0, i, 0)),
        out_shape=jax.ShapeDtypeStruct((batch, seq, hidden), concat_embeddings.dtype),
    )(concat_embeddings, table)


# speedup vs baseline: 3.4248x; 1.0839x over previous
import jax
import jax.numpy as jnp
from jax.experimental import pallas as pl


def _add_body(concat_ref, table_ref, out_ref):
    out_ref[...] = concat_ref[...] + table_ref[...][None, :, :]


def kernel(concat_embeddings, position_table):
    batch, seq, hidden = concat_embeddings.shape
    bs = 256
    grid = (seq // bs,)
    table = position_table[:seq]
    return pl.pallas_call(
        _add_body,
        grid=grid,
        in_specs=[
            pl.BlockSpec((batch, bs, hidden), lambda i: (0, i, 0)),
            pl.BlockSpec((bs, hidden), lambda i: (i, 0)),
        ],
        out_specs=pl.BlockSpec((batch, bs, hidden), lambda i: (0, i, 0)),
        out_shape=jax.ShapeDtypeStruct((batch, seq, hidden), concat_embeddings.dtype),
    )(concat_embeddings, table)
